# baseline (speedup 1.0000x reference)
import functools

import jax
import jax.numpy as jnp
from jax import lax
from jax.experimental import pallas as pl
from jax.experimental.pallas import tpu as pltpu
from jax.experimental.pallas import tpu_sc as plsc

D_MODEL_ = 1024
WINDOW_ = 4096
BATCH_ = 4

NC_ = 2
NS_ = 16
NW_ = NC_ * NS_
LANES_ = 16

ROWS_PER_W_ = WINDOW_ // NW_
RCH_ = 16
CH_ = RCH_ * D_MODEL_
NCHUNK_ = ROWS_PER_W_ // RCH_
NITEMS_ = NCHUNK_ * BATCH_
NXB_ = 4
KAHEAD_ = 2
GROUP_ = 8


def _sc_body(x_hbm, emb_hbm, out_hbm, *scratch):
    xbufs = scratch[0:NXB_]
    ebufs = scratch[NXB_:NXB_ + 2]
    xsems = scratch[NXB_ + 2:2 * NXB_ + 2]
    osems = scratch[2 * NXB_ + 2:3 * NXB_ + 2]
    esems = scratch[3 * NXB_ + 2:3 * NXB_ + 4]

    wid = lax.axis_index("s") * NC_ + lax.axis_index("c")
    row0 = wid * ROWS_PER_W_

    def xload_desc(i, b, slot):
        t = i >> 2
        return pltpu.make_async_copy(
            x_hbm.at[b, pl.ds(row0 + t * RCH_, RCH_)], xbufs[slot], xsems[slot]
        )

    def store_desc(i, b, slot):
        t = i >> 2
        return pltpu.make_async_copy(
            xbufs[slot], out_hbm.at[b, pl.ds(row0 + t * RCH_, RCH_)], osems[slot]
        )

    def eload_desc(t, par):
        return pltpu.make_async_copy(
            emb_hbm.at[pl.ds(row0 + t * RCH_, RCH_)], ebufs[par], esems[par]
        )

    eload_desc(0, 0).start()
    for i in range(KAHEAD_):
        xload_desc(i, i & 3, i % NXB_).start()

    def group(k, _):
        i0 = k * GROUP_
        for p in range(GROUP_):
            i = i0 + p
            b = p & 3
            slot = p % NXB_
            par = (p >> 2) & 1
            t = i >> 2

            if p == 1:
                eload_desc(t + 1, par ^ 1).start()
            elif p == 5:

                @pl.when(i0 < NITEMS_ - GROUP_)
                def _():
                    eload_desc(t + 1, par ^ 1).start()

            if b == 0:
                eload_desc(t, par).wait()

            xload_desc(i, b, slot).wait()

            if p >= KAHEAD_:
                store_desc(i - KAHEAD_, (p - KAHEAD_) & 3,
                           (p - KAHEAD_) % NXB_).wait()
            else:

                @pl.when(i0 >= KAHEAD_ - p)
                def _():
                    store_desc(i - KAHEAD_, (p - KAHEAD_) & 3,
                               (p - KAHEAD_) % NXB_).wait()

            if p < GROUP_ - KAHEAD_:
                xload_desc(i + KAHEAD_, (p + KAHEAD_) & 3,
                           (p + KAHEAD_) % NXB_).start()
            else:

                @pl.when(i0 < NITEMS_ - GROUP_)
                def _():
                    xload_desc(i + KAHEAD_, (p + KAHEAD_) & 3,
                               (p + KAHEAD_) % NXB_).start()

            xbuf = xbufs[slot]
            ebuf = ebufs[par]

            @plsc.parallel_loop(0, CH_ // LANES_, 1, unroll=16)
            def _add_loop(j, xbuf=xbuf, ebuf=ebuf):
                r = j >> 6
                s = pl.ds((j & 63) * LANES_, LANES_)
                xbuf[r, s] = xbuf[r, s] + ebuf[r, s]

            store_desc(i, b, slot).start()
        return 0

    lax.fori_loop(0, NITEMS_ // GROUP_, group, 0)

    for i in range(NITEMS_ - KAHEAD_, NITEMS_):
        p = i % GROUP_
        store_desc(i, p & 3, p % NXB_).wait()


_sc_add = functools.partial(
    pl.kernel,
    out_type=jax.ShapeDtypeStruct((BATCH_, WINDOW_, D_MODEL_), jnp.float32),
    mesh=plsc.VectorSubcoreMesh(
        core_axis_name="c", subcore_axis_name="s", num_cores=NC_, num_subcores=NS_
    ),
    scratch_types=(
        [pltpu.VMEM((RCH_, D_MODEL_), jnp.float32)] * NXB_
        + [pltpu.VMEM((RCH_, D_MODEL_), jnp.float32)] * 2
        + [pltpu.SemaphoreType.DMA] * (2 * NXB_ + 2)
    ),
)(_sc_body)


def kernel(X, emb):
    return _sc_add(X, emb)
